# baseline (speedup 1.0000x reference)
import functools

import jax
import jax.numpy as jnp
from jax import lax
from jax.experimental import pallas as pl
from jax.experimental.pallas import tpu as pltpu
from jax.experimental.pallas import tpu_sc as plsc

NL = 8
K = 8192
D = 256
T = 4096
TM = 1024
KB = 8192
NKB = K // KB

H = T // 2
NW = 32
BPW = H // NW


def _argmin_block(k, r, w_ref, x2h_ref, y2h_ref, macc_ref, iacc_ref, idx_ref):
    w = w_ref[...]

    @pl.when(pl.program_id(0) == 0)
    def _y2():
        y2h_ref[pl.ds(k * KB, KB), :] = jnp.sum(w * w, axis=1,
                                                keepdims=True) * 0.5

    @pl.when(k == 0)
    def _init():
        macc_ref[...] = jnp.full((8, TM), jnp.inf, dtype=jnp.float32)
        iacc_ref[...] = jnp.zeros((8, TM), dtype=jnp.float32)

    s = lax.dot_general(w, r, (((1,), (1,)), ((), ())),
                        preferred_element_type=jnp.float32)
    x2h = x2h_ref[...]
    macc = macc_ref[...]
    iacc = iacc_ref[...]
    base = lax.convert_element_type(k * (KB // 8), jnp.float32)
    for i in range(KB // 8):
        y2i = y2h_ref[pl.ds(k * KB + i * 8, 8), :]
        di = (x2h + y2i) - s[i * 8:(i + 1) * 8, :]
        mask = di < macc
        iacc = jnp.where(mask, base + float(i), iacc)
        macc = jnp.where(mask, di, macc)
    macc_ref[...] = macc
    iacc_ref[...] = iacc

    @pl.when(k == NKB - 1)
    def _flush():
        subl = lax.broadcasted_iota(jnp.int32, (8, TM), 0).astype(jnp.float32)
        kv = iacc * 8.0 + subl
        m = jnp.min(macc, axis=0, keepdims=True)
        loc = jnp.min(jnp.where(macc == m, kv, float(K)), axis=0,
                      keepdims=True)
        idx_ref[...] = loc.astype(jnp.int32).reshape(1, 1, TM)


def _store_x2h(r, x2h_ref):
    x2col = jnp.sum(r * r, axis=1, keepdims=True) * 0.5
    x2h_ref[...] = jnp.transpose(x2col, (1, 0))


def layer0_body(r_ref, w_ref, idx_ref, x2h_ref, y2h_ref, macc_ref, iacc_ref):
    k = pl.program_id(1)

    @pl.when(k == 0)
    def _init():
        _store_x2h(r_ref[...], x2h_ref)

    _argmin_block(k, r_ref[...], w_ref, x2h_ref, y2h_ref, macc_ref, iacc_ref,
                  idx_ref)


def fused_body(rprev_ref, qprev_ref, zqprev_ref, w_ref,
               idx_ref, rnew_ref, zqnew_ref, lsum_ref,
               x2h_ref, y2h_ref, macc_ref, iacc_ref, lacc_ref):
    t = pl.program_id(0)
    k = pl.program_id(1)

    @pl.when(k == 0)
    def _update():
        rp = rprev_ref[...]
        q = qprev_ref[...]
        diff = rp - q
        part = jnp.sum(diff * diff)

        @pl.when(t == 0)
        def _l0():
            lacc_ref[0] = part

        @pl.when(t > 0)
        def _ln():
            lacc_ref[0] = lacc_ref[0] + part

        qs = rp + (q - rp)
        zqnew_ref[...] = zqprev_ref[...] + qs
        r = rp - qs
        rnew_ref[...] = r
        _store_x2h(r, x2h_ref)

    @pl.when((t == H // TM - 1) & (k == NKB - 1))
    def _lout():
        lsum_ref[0, 0] = lacc_ref[0]

    _argmin_block(k, rnew_ref[...], w_ref, x2h_ref, y2h_ref, macc_ref,
                  iacc_ref, idx_ref)


_scratch = [
    pltpu.VMEM((1, TM), jnp.float32),
    pltpu.VMEM((K, 1), jnp.float32),
    pltpu.VMEM((8, TM), jnp.float32),
    pltpu.VMEM((8, TM), jnp.float32),
]

_params = pltpu.CompilerParams(dimension_semantics=("arbitrary", "arbitrary"))

_IDX_SHAPE = jax.ShapeDtypeStruct((H // TM, 1, TM), jnp.int32)
_idx_spec = pl.BlockSpec((1, 1, TM), lambda t, k: (t, 0, 0))


def _make_layer0(toff):
    return pl.pallas_call(
        layer0_body,
        grid=(H // TM, NKB),
        in_specs=[
            pl.BlockSpec((TM, D), lambda t, k: (t + toff, 0)),
            pl.BlockSpec((KB, D), lambda t, k: (k, 0)),
        ],
        out_specs=_idx_spec,
        out_shape=_IDX_SHAPE,
        scratch_shapes=_scratch,
        compiler_params=_params,
    )


layer0_calls = [_make_layer0(0), _make_layer0(H // TM)]

fused_call = pl.pallas_call(
    fused_body,
    grid=(H // TM, NKB),
    in_specs=[
        pl.BlockSpec((TM, D), lambda t, k: (t, 0)),
        pl.BlockSpec((TM, D), lambda t, k: (t, 0)),
        pl.BlockSpec((TM, D), lambda t, k: (t, 0)),
        pl.BlockSpec((KB, D), lambda t, k: (k, 0)),
    ],
    out_specs=[
        _idx_spec,
        pl.BlockSpec((TM, D), lambda t, k: (t, 0)),
        pl.BlockSpec((TM, D), lambda t, k: (t, 0)),
        pl.BlockSpec(memory_space=pltpu.SMEM),
    ],
    out_shape=[
        _IDX_SHAPE,
        jax.ShapeDtypeStruct((H, D), jnp.float32),
        jax.ShapeDtypeStruct((H, D), jnp.float32),
        jax.ShapeDtypeStruct((1, 1), jnp.float32),
    ],
    scratch_shapes=_scratch + [pltpu.SMEM((1,), jnp.float32)],
    compiler_params=_params,
)


@functools.lru_cache(maxsize=None)
def _sc_gather(off):
    @functools.partial(
        pl.kernel,
        mesh=plsc.VectorSubcoreMesh(core_axis_name="c", subcore_axis_name="s"),
        out_type=jax.ShapeDtypeStruct((H, D), jnp.float32),
        scratch_types=[
            pltpu.VMEM((BPW,), jnp.int32),
            pltpu.VMEM((BPW, D), jnp.float32),
            pltpu.SemaphoreType.DMA,
        ],
    )
    def sc_gather(cb_hbm, gidx_hbm, out_hbm, idx_v, rows_v, sem):
        wid = lax.axis_index("s") * 2 + lax.axis_index("c")
        base = wid * BPW
        pltpu.sync_copy(gidx_hbm.at[pl.ds(base, BPW)], idx_v)
        if off:
            for j in range(BPW // 16):
                sl = pl.ds(j * 16, 16)
                idx_v[sl] = idx_v[sl] + off
        pltpu.async_copy(cb_hbm.at[idx_v], rows_v, sem).wait()
        pltpu.sync_copy(rows_v, out_hbm.at[pl.ds(base, BPW)])

    return sc_gather


def kernel(z, codebooks):
    batch, dim, time = z.shape
    zt = jnp.transpose(z, (0, 2, 1))
    r0 = zt.reshape(T, D)
    cb_flat = codebooks.reshape(NL * K, D)

    idx = [layer0_calls[h](r0, codebooks[0]).reshape(H) for h in range(2)]
    q = [_sc_gather(0)(cb_flat, idx[h]) for h in range(2)]
    codes = [idx]
    r = [r0[:H], r0[H:]]
    zq = [jnp.zeros((H, D), dtype=jnp.float32) for _ in range(2)]
    loss = jnp.zeros((), dtype=jnp.float32)
    inv_n = jnp.float32(1.0 / (T * D))
    for layer in range(1, NL):
        idx, lsum = [None, None], [None, None]
        for h in range(2):
            idx[h], r[h], zq[h], lsum[h] = fused_call(
                r[h], q[h], zq[h], codebooks[layer])
            idx[h] = idx[h].reshape(H)
        for h in range(2):
            q[h] = _sc_gather(layer * K)(cb_flat, idx[h])
        loss = loss + (lsum[0][0, 0] + lsum[1][0, 0]) * inv_n
        codes.append(idx)

    loss = loss + (jnp.sum((r[0] - q[0]) ** 2)
                   + jnp.sum((r[1] - q[1]) ** 2)) * inv_n
    zq_full = jnp.concatenate(
        [zqh + (rh + (qh - rh)) for zqh, rh, qh in zip(zq, r, q)], axis=0)

    z_q_out = jnp.transpose(zq_full.reshape(batch, time, dim), (0, 2, 1))
    all_codes = jnp.stack(
        [jnp.concatenate(c, axis=0).reshape(batch, time) for c in codes],
        axis=0)
    return (z_q_out, all_codes, loss, loss, loss + loss)


# speedup vs baseline: 1.1391x; 1.1391x over previous
import functools

import jax
import jax.numpy as jnp
from jax import lax
from jax.experimental import pallas as pl
from jax.experimental.pallas import tpu as pltpu
from jax.experimental.pallas import tpu_sc as plsc

NL = 8
K = 8192
D = 256
T = 4096
TM = 4096
KB = 8192
NKB = K // KB

NW = 32
BPW = T // NW


def _argmin_block(k, r, w_ref, x2h_ref, y2h_ref, macc_ref, iacc_ref, idx_ref):
    w = w_ref[...]

    @pl.when(pl.program_id(0) == 0)
    def _y2():
        y2h_ref[pl.ds(k * KB, KB), :] = jnp.sum(w * w, axis=1,
                                                keepdims=True) * 0.5

    @pl.when(k == 0)
    def _init():
        macc_ref[...] = jnp.full((8, TM), jnp.inf, dtype=jnp.float32)
        iacc_ref[...] = jnp.zeros((8, TM), dtype=jnp.float32)

    s = lax.dot_general(w, r, (((1,), (1,)), ((), ())),
                        preferred_element_type=jnp.float32)
    x2h = x2h_ref[...]
    macc = macc_ref[...]
    iacc = iacc_ref[...]
    base = lax.convert_element_type(k * (KB // 8), jnp.float32)
    for i in range(KB // 8):
        y2i = y2h_ref[pl.ds(k * KB + i * 8, 8), :]
        di = (x2h + y2i) - s[i * 8:(i + 1) * 8, :]
        mask = di < macc
        iacc = jnp.where(mask, base + float(i), iacc)
        macc = jnp.where(mask, di, macc)
    macc_ref[...] = macc
    iacc_ref[...] = iacc

    @pl.when(k == NKB - 1)
    def _flush():
        subl = lax.broadcasted_iota(jnp.int32, (8, TM), 0).astype(jnp.float32)
        kv = iacc * 8.0 + subl
        m = jnp.min(macc, axis=0, keepdims=True)
        loc = jnp.min(jnp.where(macc == m, kv, float(K)), axis=0,
                      keepdims=True)
        idx_ref[...] = loc.astype(jnp.int32).reshape(1, 1, TM)


def _store_x2h(r, x2h_ref):
    x2col = jnp.sum(r * r, axis=1, keepdims=True) * 0.5
    x2h_ref[...] = jnp.transpose(x2col, (1, 0))


def layer0_body(r_ref, w_ref, idx_ref, x2h_ref, y2h_ref, macc_ref, iacc_ref):
    k = pl.program_id(1)

    @pl.when(k == 0)
    def _init():
        _store_x2h(r_ref[...], x2h_ref)

    _argmin_block(k, r_ref[...], w_ref, x2h_ref, y2h_ref, macc_ref, iacc_ref,
                  idx_ref)


def fused_body(rprev_ref, qprev_ref, zqprev_ref, w_ref,
               idx_ref, rnew_ref, zqnew_ref, lsum_ref,
               x2h_ref, y2h_ref, macc_ref, iacc_ref, lacc_ref):
    t = pl.program_id(0)
    k = pl.program_id(1)

    @pl.when(k == 0)
    def _update():
        rp = rprev_ref[...]
        q = qprev_ref[...]
        diff = rp - q
        part = jnp.sum(diff * diff)

        @pl.when(t == 0)
        def _l0():
            lacc_ref[0] = part

        @pl.when(t > 0)
        def _ln():
            lacc_ref[0] = lacc_ref[0] + part

        qs = rp + (q - rp)
        zqnew_ref[...] = zqprev_ref[...] + qs
        r = rp - qs
        rnew_ref[...] = r
        _store_x2h(r, x2h_ref)

    @pl.when((t == T // TM - 1) & (k == NKB - 1))
    def _lout():
        lsum_ref[0, 0] = lacc_ref[0]

    _argmin_block(k, rnew_ref[...], w_ref, x2h_ref, y2h_ref, macc_ref,
                  iacc_ref, idx_ref)


_scratch = [
    pltpu.VMEM((1, TM), jnp.float32),
    pltpu.VMEM((K, 1), jnp.float32),
    pltpu.VMEM((8, TM), jnp.float32),
    pltpu.VMEM((8, TM), jnp.float32),
]

_params = pltpu.CompilerParams(dimension_semantics=("arbitrary", "arbitrary"))

_IDX_SHAPE = jax.ShapeDtypeStruct((T // TM, 1, TM), jnp.int32)
_idx_spec = pl.BlockSpec((1, 1, TM), lambda t, k: (t, 0, 0))

layer0_call = pl.pallas_call(
    layer0_body,
    grid=(T // TM, NKB),
    in_specs=[
        pl.BlockSpec((TM, D), lambda t, k: (t, 0)),
        pl.BlockSpec((KB, D), lambda t, k: (k, 0)),
    ],
    out_specs=_idx_spec,
    out_shape=_IDX_SHAPE,
    scratch_shapes=_scratch,
    compiler_params=_params,
)

fused_call = pl.pallas_call(
    fused_body,
    grid=(T // TM, NKB),
    in_specs=[
        pl.BlockSpec((TM, D), lambda t, k: (t, 0)),
        pl.BlockSpec((TM, D), lambda t, k: (t, 0)),
        pl.BlockSpec((TM, D), lambda t, k: (t, 0)),
        pl.BlockSpec((KB, D), lambda t, k: (k, 0)),
    ],
    out_specs=[
        _idx_spec,
        pl.BlockSpec((TM, D), lambda t, k: (t, 0)),
        pl.BlockSpec((TM, D), lambda t, k: (t, 0)),
        pl.BlockSpec(memory_space=pltpu.SMEM),
    ],
    out_shape=[
        _IDX_SHAPE,
        jax.ShapeDtypeStruct((T, D), jnp.float32),
        jax.ShapeDtypeStruct((T, D), jnp.float32),
        jax.ShapeDtypeStruct((1, 1), jnp.float32),
    ],
    scratch_shapes=_scratch + [pltpu.SMEM((1,), jnp.float32)],
    compiler_params=_params,
)


@functools.lru_cache(maxsize=None)
def _sc_gather(off):
    @functools.partial(
        pl.kernel,
        mesh=plsc.VectorSubcoreMesh(core_axis_name="c", subcore_axis_name="s"),
        out_type=jax.ShapeDtypeStruct((T, D), jnp.float32),
        scratch_types=[
            pltpu.VMEM((BPW,), jnp.int32),
            pltpu.VMEM((BPW, D), jnp.float32),
            pltpu.SemaphoreType.DMA,
        ],
    )
    def sc_gather(cb_hbm, gidx_hbm, out_hbm, idx_v, rows_v, sem):
        wid = lax.axis_index("s") * 2 + lax.axis_index("c")
        base = wid * BPW
        pltpu.sync_copy(gidx_hbm.at[pl.ds(base, BPW)], idx_v)
        if off:
            for j in range(BPW // 16):
                sl = pl.ds(j * 16, 16)
                idx_v[sl] = idx_v[sl] + off
        pltpu.async_copy(cb_hbm.at[idx_v], rows_v, sem).wait()
        pltpu.sync_copy(rows_v, out_hbm.at[pl.ds(base, BPW)])

    return sc_gather


def kernel(z, codebooks):
    batch, dim, time = z.shape
    zt = jnp.transpose(z, (0, 2, 1))
    r0 = zt.reshape(T, D)
    cb_flat = codebooks.reshape(NL * K, D)
    idx = layer0_call(r0, codebooks[0]).reshape(T)
    q = _sc_gather(0)(cb_flat, idx)
    codes = [idx]
    r, zq = r0, jnp.zeros_like(r0)
    loss = jnp.zeros((), dtype=jnp.float32)
    inv_n = jnp.float32(1.0 / (T * D))
    for layer in range(1, NL):
        idx, r, zq, lsum = fused_call(r, q, zq, codebooks[layer])
        loss = loss + lsum[0, 0] * inv_n
        idx = idx.reshape(T)
        q = _sc_gather(layer * K)(cb_flat, idx)
        codes.append(idx)

    loss = loss + jnp.mean((r - q) ** 2)
    qs = r + (q - r)
    zq = zq + qs

    z_q_out = jnp.transpose(zq.reshape(batch, time, dim), (0, 2, 1))
    all_codes = jnp.stack([c.reshape(batch, time) for c in codes], axis=0)
    return (z_q_out, all_codes, loss, loss, loss + loss)


# speedup vs baseline: 1.1424x; 1.0029x over previous
import functools

import jax
import jax.numpy as jnp
from jax import lax
from jax.experimental import pallas as pl
from jax.experimental.pallas import tpu as pltpu
from jax.experimental.pallas import tpu_sc as plsc

NL = 8
K = 8192
D = 256
T = 4096
TM = 1024
KB = 8192
NKB = K // KB

NW = 32
BPW = T // NW


def _argmin_block(k, r, w_ref, x2h_ref, y2h_ref, macc_ref, iacc_ref, idx_ref):
    w = w_ref[...]

    @pl.when(pl.program_id(0) == 0)
    def _y2():
        y2h_ref[pl.ds(k * KB, KB), :] = jnp.sum(w * w, axis=1,
                                                keepdims=True) * 0.5

    @pl.when(k == 0)
    def _init():
        macc_ref[...] = jnp.full((16, TM), jnp.inf, dtype=jnp.float32)
        iacc_ref[...] = jnp.zeros((16, TM), dtype=jnp.float32)

    s = lax.dot_general(w, r, (((1,), (1,)), ((), ())),
                        preferred_element_type=jnp.float32)
    x2h = x2h_ref[...]
    macc = [macc_ref[0:8, :], macc_ref[8:16, :]]
    iacc = [iacc_ref[0:8, :], iacc_ref[8:16, :]]
    base = lax.convert_element_type(k * (KB // 8), jnp.float32)
    for j in range(KB // 16):
        for p in range(2):
            i = 2 * j + p
            y2i = y2h_ref[pl.ds(k * KB + i * 8, 8), :]
            di = (x2h + y2i) - s[i * 8:(i + 1) * 8, :]
            mask = di < macc[p]
            iacc[p] = jnp.where(mask, base + float(i), iacc[p])
            macc[p] = jnp.where(mask, di, macc[p])
    macc_ref[0:8, :] = macc[0]
    macc_ref[8:16, :] = macc[1]
    iacc_ref[0:8, :] = iacc[0]
    iacc_ref[8:16, :] = iacc[1]

    @pl.when(k == NKB - 1)
    def _flush():
        subl = lax.broadcasted_iota(jnp.int32, (8, TM), 0).astype(jnp.float32)
        kv = [iacc[p] * 8.0 + subl for p in range(2)]
        m = jnp.minimum(jnp.min(macc[0], axis=0, keepdims=True),
                        jnp.min(macc[1], axis=0, keepdims=True))
        loc = jnp.minimum(
            jnp.min(jnp.where(macc[0] == m, kv[0], float(K)), axis=0,
                    keepdims=True),
            jnp.min(jnp.where(macc[1] == m, kv[1], float(K)), axis=0,
                    keepdims=True))
        idx_ref[...] = loc.astype(jnp.int32).reshape(1, 1, TM)


def _store_x2h(r, x2h_ref):
    x2col = jnp.sum(r * r, axis=1, keepdims=True) * 0.5
    x2h_ref[...] = jnp.transpose(x2col, (1, 0))


def layer0_body(r_ref, w_ref, idx_ref, x2h_ref, y2h_ref, macc_ref, iacc_ref):
    k = pl.program_id(1)

    @pl.when(k == 0)
    def _init():
        _store_x2h(r_ref[...], x2h_ref)

    _argmin_block(k, r_ref[...], w_ref, x2h_ref, y2h_ref, macc_ref, iacc_ref,
                  idx_ref)


def fused_body(rprev_ref, qprev_ref, zqprev_ref, w_ref,
               idx_ref, rnew_ref, zqnew_ref, lsum_ref,
               x2h_ref, y2h_ref, macc_ref, iacc_ref, lacc_ref):
    t = pl.program_id(0)
    k = pl.program_id(1)

    @pl.when(k == 0)
    def _update():
        rp = rprev_ref[...]
        q = qprev_ref[...]
        diff = rp - q
        part = jnp.sum(diff * diff)

        @pl.when(t == 0)
        def _l0():
            lacc_ref[0] = part

        @pl.when(t > 0)
        def _ln():
            lacc_ref[0] = lacc_ref[0] + part

        qs = rp + (q - rp)
        zqnew_ref[...] = zqprev_ref[...] + qs
        r = rp - qs
        rnew_ref[...] = r
        _store_x2h(r, x2h_ref)

    @pl.when((t == T // TM - 1) & (k == NKB - 1))
    def _lout():
        lsum_ref[0, 0] = lacc_ref[0]

    _argmin_block(k, rnew_ref[...], w_ref, x2h_ref, y2h_ref, macc_ref,
                  iacc_ref, idx_ref)


_scratch = [
    pltpu.VMEM((1, TM), jnp.float32),
    pltpu.VMEM((K, 1), jnp.float32),
    pltpu.VMEM((16, TM), jnp.float32),
    pltpu.VMEM((16, TM), jnp.float32),
]

_params = pltpu.CompilerParams(dimension_semantics=("arbitrary", "arbitrary"))

_IDX_SHAPE = jax.ShapeDtypeStruct((T // TM, 1, TM), jnp.int32)
_idx_spec = pl.BlockSpec((1, 1, TM), lambda t, k: (t, 0, 0))

layer0_call = pl.pallas_call(
    layer0_body,
    grid=(T // TM, NKB),
    in_specs=[
        pl.BlockSpec((TM, D), lambda t, k: (t, 0)),
        pl.BlockSpec((KB, D), lambda t, k: (k, 0)),
    ],
    out_specs=_idx_spec,
    out_shape=_IDX_SHAPE,
    scratch_shapes=_scratch,
    compiler_params=_params,
)

fused_call = pl.pallas_call(
    fused_body,
    grid=(T // TM, NKB),
    in_specs=[
        pl.BlockSpec((TM, D), lambda t, k: (t, 0)),
        pl.BlockSpec((TM, D), lambda t, k: (t, 0)),
        pl.BlockSpec((TM, D), lambda t, k: (t, 0)),
        pl.BlockSpec((KB, D), lambda t, k: (k, 0)),
    ],
    out_specs=[
        _idx_spec,
        pl.BlockSpec((TM, D), lambda t, k: (t, 0)),
        pl.BlockSpec((TM, D), lambda t, k: (t, 0)),
        pl.BlockSpec(memory_space=pltpu.SMEM),
    ],
    out_shape=[
        _IDX_SHAPE,
        jax.ShapeDtypeStruct((T, D), jnp.float32),
        jax.ShapeDtypeStruct((T, D), jnp.float32),
        jax.ShapeDtypeStruct((1, 1), jnp.float32),
    ],
    scratch_shapes=_scratch + [pltpu.SMEM((1,), jnp.float32)],
    compiler_params=_params,
)


@functools.lru_cache(maxsize=None)
def _sc_gather(off):
    @functools.partial(
        pl.kernel,
        mesh=plsc.VectorSubcoreMesh(core_axis_name="c", subcore_axis_name="s"),
        out_type=jax.ShapeDtypeStruct((T, D), jnp.float32),
        scratch_types=[
            pltpu.VMEM((BPW,), jnp.int32),
            pltpu.VMEM((BPW, D), jnp.float32),
            pltpu.SemaphoreType.DMA,
        ],
    )
    def sc_gather(cb_hbm, gidx_hbm, out_hbm, idx_v, rows_v, sem):
        wid = lax.axis_index("s") * 2 + lax.axis_index("c")
        base = wid * BPW
        pltpu.sync_copy(gidx_hbm.at[pl.ds(base, BPW)], idx_v)
        if off:
            for j in range(BPW // 16):
                sl = pl.ds(j * 16, 16)
                idx_v[sl] = idx_v[sl] + off
        pltpu.async_copy(cb_hbm.at[idx_v], rows_v, sem).wait()
        pltpu.sync_copy(rows_v, out_hbm.at[pl.ds(base, BPW)])

    return sc_gather


def kernel(z, codebooks):
    batch, dim, time = z.shape
    zt = jnp.transpose(z, (0, 2, 1))
    r0 = zt.reshape(T, D)
    cb_flat = codebooks.reshape(NL * K, D)
    idx = layer0_call(r0, codebooks[0]).reshape(T)
    q = _sc_gather(0)(cb_flat, idx)
    codes = [idx]
    r, zq = r0, jnp.zeros_like(r0)
    loss = jnp.zeros((), dtype=jnp.float32)
    inv_n = jnp.float32(1.0 / (T * D))
    for layer in range(1, NL):
        idx, r, zq, lsum = fused_call(r, q, zq, codebooks[layer])
        loss = loss + lsum[0, 0] * inv_n
        idx = idx.reshape(T)
        q = _sc_gather(layer * K)(cb_flat, idx)
        codes.append(idx)

    loss = loss + jnp.mean((r - q) ** 2)
    qs = r + (q - r)
    zq = zq + qs

    z_q_out = jnp.transpose(zq.reshape(batch, time, dim), (0, 2, 1))
    all_codes = jnp.stack([c.reshape(batch, time) for c in codes], axis=0)
    return (z_q_out, all_codes, loss, loss, loss + loss)


# speedup vs baseline: 1.1542x; 1.0103x over previous
import functools

import jax
import jax.numpy as jnp
from jax import lax
from jax.experimental import pallas as pl
from jax.experimental.pallas import tpu as pltpu
from jax.experimental.pallas import tpu_sc as plsc

NL = 8
K = 8192
D = 256
T = 4096
TM = 1024
KB = 8192
NKB = K // KB

NW = 32
BPW = T // NW


def _argmin_block(k, r, w_ref, x2h_ref, y2h_ref, macc_ref, iacc_ref, idx_ref):
    w = w_ref[...]

    @pl.when(pl.program_id(0) == 0)
    def _y2():
        y2h_ref[pl.ds(k * KB, KB), :] = jnp.sum(w * w, axis=1,
                                                keepdims=True) * 0.5

    @pl.when(k == 0)
    def _init():
        macc_ref[...] = jnp.full((8, TM), jnp.inf, dtype=jnp.float32)
        iacc_ref[...] = jnp.zeros((8, TM), dtype=jnp.float32)

    s = lax.dot_general(w, r, (((1,), (1,)), ((), ())),
                        preferred_element_type=jnp.float32)
    x2h = x2h_ref[...]
    macc = macc_ref[...]
    iacc = iacc_ref[...]
    base = lax.convert_element_type(k * (KB // 8), jnp.float32)
    for i in range(KB // 8):
        y2i = y2h_ref[pl.ds(k * KB + i * 8, 8), :]
        di = (x2h + y2i) - s[i * 8:(i + 1) * 8, :]
        mask = di < macc
        iacc = jnp.where(mask, base + float(i), iacc)
        macc = jnp.where(mask, di, macc)
    macc_ref[...] = macc
    iacc_ref[...] = iacc

    @pl.when(k == NKB - 1)
    def _flush():
        subl = lax.broadcasted_iota(jnp.int32, (8, TM), 0).astype(jnp.float32)
        kv = iacc * 8.0 + subl
        m = jnp.min(macc, axis=0, keepdims=True)
        loc = jnp.min(jnp.where(macc == m, kv, float(K)), axis=0,
                      keepdims=True)
        idx_ref[...] = loc.astype(jnp.int32).reshape(TM)


def _store_x2h(r, x2h_ref):
    x2col = jnp.sum(r * r, axis=1, keepdims=True) * 0.5
    x2h_ref[...] = jnp.transpose(x2col, (1, 0))


def layer0_body(r_ref, w_ref, idx_ref, x2h_ref, y2h_ref, macc_ref, iacc_ref):
    k = pl.program_id(1)

    @pl.when(k == 0)
    def _init():
        _store_x2h(r_ref[...], x2h_ref)

    _argmin_block(k, r_ref[...], w_ref, x2h_ref, y2h_ref, macc_ref, iacc_ref,
                  idx_ref)


def fused_body(rprev_ref, qprev_ref, zqprev_ref, w_ref,
               idx_ref, rnew_ref, zqnew_ref, lsum_ref,
               x2h_ref, y2h_ref, macc_ref, iacc_ref, lacc_ref):
    t = pl.program_id(0)
    k = pl.program_id(1)

    @pl.when(k == 0)
    def _update():
        rp = rprev_ref[...]
        q = qprev_ref[...]
        diff = rp - q
        part = jnp.sum(diff * diff)

        @pl.when(t == 0)
        def _l0():
            lacc_ref[0] = part

        @pl.when(t > 0)
        def _ln():
            lacc_ref[0] = lacc_ref[0] + part

        qs = rp + (q - rp)
        zqnew_ref[...] = zqprev_ref[...] + qs
        r = rp - qs
        rnew_ref[...] = r
        _store_x2h(r, x2h_ref)

    @pl.when((t == T // TM - 1) & (k == NKB - 1))
    def _lout():
        lsum_ref[0, 0] = lacc_ref[0]

    _argmin_block(k, rnew_ref[...], w_ref, x2h_ref, y2h_ref, macc_ref,
                  iacc_ref, idx_ref)


_scratch = [
    pltpu.VMEM((1, TM), jnp.float32),
    pltpu.VMEM((K, 1), jnp.float32),
    pltpu.VMEM((8, TM), jnp.float32),
    pltpu.VMEM((8, TM), jnp.float32),
]

_params = pltpu.CompilerParams(dimension_semantics=("arbitrary", "arbitrary"))

_IDX_SHAPE = jax.ShapeDtypeStruct((T,), jnp.int32)
_idx_spec = pl.BlockSpec((TM,), lambda t, k: (t,))

layer0_call = pl.pallas_call(
    layer0_body,
    grid=(T // TM, NKB),
    in_specs=[
        pl.BlockSpec((TM, D), lambda t, k: (t, 0)),
        pl.BlockSpec((KB, D), lambda t, k: (k, 0)),
    ],
    out_specs=_idx_spec,
    out_shape=_IDX_SHAPE,
    scratch_shapes=_scratch,
    compiler_params=_params,
)

fused_call = pl.pallas_call(
    fused_body,
    grid=(T // TM, NKB),
    in_specs=[
        pl.BlockSpec((TM, D), lambda t, k: (t, 0)),
        pl.BlockSpec((TM, D), lambda t, k: (t, 0)),
        pl.BlockSpec((TM, D), lambda t, k: (t, 0)),
        pl.BlockSpec((KB, D), lambda t, k: (k, 0)),
    ],
    out_specs=[
        _idx_spec,
        pl.BlockSpec((TM, D), lambda t, k: (t, 0)),
        pl.BlockSpec((TM, D), lambda t, k: (t, 0)),
        pl.BlockSpec(memory_space=pltpu.SMEM),
    ],
    out_shape=[
        _IDX_SHAPE,
        jax.ShapeDtypeStruct((T, D), jnp.float32),
        jax.ShapeDtypeStruct((T, D), jnp.float32),
        jax.ShapeDtypeStruct((1, 1), jnp.float32),
    ],
    scratch_shapes=_scratch + [pltpu.SMEM((1,), jnp.float32)],
    compiler_params=_params,
)


@functools.lru_cache(maxsize=None)
def _sc_gather(off):
    @functools.partial(
        pl.kernel,
        mesh=plsc.VectorSubcoreMesh(core_axis_name="c", subcore_axis_name="s"),
        out_type=jax.ShapeDtypeStruct((T, D), jnp.float32),
        scratch_types=[
            pltpu.VMEM((BPW,), jnp.int32),
            pltpu.VMEM((BPW, D), jnp.float32),
            pltpu.SemaphoreType.DMA,
        ],
    )
    def sc_gather(cb_hbm, gidx_hbm, out_hbm, idx_v, rows_v, sem):
        wid = lax.axis_index("s") * 2 + lax.axis_index("c")
        base = wid * BPW
        pltpu.sync_copy(gidx_hbm.at[pl.ds(base, BPW)], idx_v)
        if off:
            for j in range(BPW // 16):
                sl = pl.ds(j * 16, 16)
                idx_v[sl] = idx_v[sl] + off
        pltpu.async_copy(cb_hbm.at[idx_v], rows_v, sem).wait()
        pltpu.sync_copy(rows_v, out_hbm.at[pl.ds(base, BPW)])

    return sc_gather


def kernel(z, codebooks):
    batch, dim, time = z.shape
    zt = jnp.transpose(z, (0, 2, 1))
    r0 = zt.reshape(T, D)
    cb_flat = codebooks.reshape(NL * K, D)
    idx = layer0_call(r0, codebooks[0])
    q = _sc_gather(0)(cb_flat, idx)
    codes = [idx]
    r, zq = r0, jnp.zeros_like(r0)
    loss = jnp.zeros((), dtype=jnp.float32)
    inv_n = jnp.float32(1.0 / (T * D))
    for layer in range(1, NL):
        idx, r, zq, lsum = fused_call(r, q, zq, codebooks[layer])
        loss = loss + lsum[0, 0] * inv_n
        q = _sc_gather(layer * K)(cb_flat, idx)
        codes.append(idx)

    loss = loss + jnp.mean((r - q) ** 2)
    qs = r + (q - r)
    zq = zq + qs

    z_q_out = jnp.transpose(zq.reshape(batch, time, dim), (0, 2, 1))
    all_codes = jnp.stack([c.reshape(batch, time) for c in codes], axis=0)
    return (z_q_out, all_codes, loss, loss, loss + loss)
